# Initial kernel scaffold; baseline (speedup 1.0000x reference)
import functools

import jax
import jax.numpy as jnp
from jax import lax
from jax.experimental import pallas as pl
from jax.experimental.pallas import tpu as pltpu
from jax.experimental.pallas import tpu_sc as plsc

B = 4096
L = 200
D = 32
OUT_DIM = 20

NC = 2
NS = 16
NW = NC * NS
BPW = B // NW
C0 = 104
C1 = L - C0

_mesh = plsc.VectorSubcoreMesh(core_axis_name="c", subcore_axis_name="s")


@functools.partial(
    pl.kernel,
    out_type=jax.ShapeDtypeStruct((B, D), jnp.float32),
    mesh=_mesh,
    scratch_types=[
        pltpu.VMEM((BPW, L), jnp.int32),
        pltpu.VMEM((BPW, D), jnp.float32),
        pltpu.VMEM((L, D), jnp.float32),
        pltpu.VMEM((L, D), jnp.float32),
        pltpu.SemaphoreType.DMA,
        pltpu.SemaphoreType.DMA,
    ],
)
def _pool(data_hbm, table_hbm, out_hbm, idx_v, out_v, rows0, rows1, sem0, sem1):
    wid = lax.axis_index("s") * NC + lax.axis_index("c")
    base = wid * BPW

    pltpu.sync_copy(data_hbm.at[pl.ds(base, BPW)], idx_v)

    def fire(i, rows, sem):
        pltpu.async_copy(
            table_hbm.at[idx_v.at[i, pl.ds(0, C0)]], rows.at[pl.ds(0, C0)], sem
        )
        pltpu.async_copy(
            table_hbm.at[idx_v.at[i, pl.ds(C0, C1)]], rows.at[pl.ds(C0, C1)], sem
        )

    def drain(i, rows, sem):
        pltpu.make_async_copy(
            table_hbm.at[idx_v.at[i, pl.ds(0, C0)]], rows.at[pl.ds(0, C0)], sem
        ).wait()
        pltpu.make_async_copy(
            table_hbm.at[idx_v.at[i, pl.ds(C0, C1)]], rows.at[pl.ds(C0, C1)], sem
        ).wait()

    def accumulate(i, rows):
        def body(l, carry):
            a0, a1 = carry
            a0 = a0 + rows[l, pl.ds(0, 16)]
            a1 = a1 + rows[l, pl.ds(16, 16)]
            return a0, a1

        z = jnp.zeros((16,), jnp.float32)
        a0, a1 = lax.fori_loop(0, L, body, (z, z))
        out_v[i, pl.ds(0, 16)] = a0
        out_v[i, pl.ds(16, 16)] = a1

    fire(0, rows0, sem0)
    fire(1, rows1, sem1)

    def loop_body(j, _):
        i = j * 2
        drain(i, rows0, sem0)

        @pl.when(i + 2 < BPW)
        def _fire0():
            fire(i + 2, rows0, sem0)

        accumulate(i, rows0)

        drain(i + 1, rows1, sem1)

        @pl.when(i + 3 < BPW)
        def _fire1():
            fire(i + 3, rows1, sem1)

        accumulate(i + 1, rows1)
        return 0

    lax.fori_loop(0, BPW // 2, loop_body, 0)

    pltpu.sync_copy(out_v, out_hbm.at[pl.ds(base, BPW)])


def _linear_body(pooled_ref, len_ref, w_ref, b_ref, out_ref):
    x = pooled_ref[...] / len_ref[...].astype(jnp.float32)
    out_ref[...] = (
        lax.dot_general(
            x, w_ref[...], (((1,), (1,)), ((), ())),
            preferred_element_type=jnp.float32,
        )
        + b_ref[...]
    )


_linear = pl.pallas_call(
    _linear_body,
    out_shape=jax.ShapeDtypeStruct((B, OUT_DIM), jnp.float32),
)


def kernel(data, length, table, W, b):
    data = data.astype(jnp.int32)
    pooled = _pool(data, table)
    return _linear(pooled, length.reshape(B, 1), W, b.reshape(1, OUT_DIM))


# speedup vs baseline: 2.1874x; 2.1874x over previous
import functools

import jax
import jax.numpy as jnp
from jax import lax
from jax.experimental import pallas as pl
from jax.experimental.pallas import tpu as pltpu
from jax.experimental.pallas import tpu_sc as plsc

B = 4096
L = 200
D = 32
OUT_DIM = 20

NC = 2
NS = 16
NW = NC * NS
BPW = B // NW
C0 = 104
C1 = L - C0

_mesh = plsc.VectorSubcoreMesh(core_axis_name="c", subcore_axis_name="s")


@functools.partial(
    pl.kernel,
    out_type=jax.ShapeDtypeStruct((B, D), jnp.float32),
    mesh=_mesh,
    scratch_types=[
        pltpu.VMEM((BPW, L), jnp.int32),
        pltpu.VMEM((BPW, D), jnp.float32),
        pltpu.VMEM((L, D), jnp.float32),
        pltpu.VMEM((L, D), jnp.float32),
        pltpu.SemaphoreType.DMA,
        pltpu.SemaphoreType.DMA,
    ],
    compiler_params=pltpu.CompilerParams(use_tc_tiling_on_sc=False),
)
def _pool(data_hbm, table_hbm, out_hbm, idx_v, out_v, rows0, rows1, sem0, sem1):
    wid = lax.axis_index("s") * NC + lax.axis_index("c")
    base = wid * BPW

    pltpu.sync_copy(data_hbm.at[pl.ds(base, BPW)], idx_v)

    def fire(i, rows, sem):
        pltpu.async_copy(
            table_hbm.at[idx_v.at[i, pl.ds(0, C0)]], rows.at[pl.ds(0, C0)], sem
        )
        pltpu.async_copy(
            table_hbm.at[idx_v.at[i, pl.ds(C0, C1)]], rows.at[pl.ds(C0, C1)], sem
        )

    def drain(i, rows, sem):
        pltpu.make_async_copy(
            table_hbm.at[idx_v.at[i, pl.ds(0, C0)]], rows.at[pl.ds(0, C0)], sem
        ).wait()
        pltpu.make_async_copy(
            table_hbm.at[idx_v.at[i, pl.ds(C0, C1)]], rows.at[pl.ds(C0, C1)], sem
        ).wait()

    def accumulate(i, rows):
        def body(l, carry):
            a0, a1 = carry
            a0 = a0 + rows[l, pl.ds(0, 16)]
            a1 = a1 + rows[l, pl.ds(16, 16)]
            return a0, a1

        z = jnp.zeros((16,), jnp.float32)
        a0, a1 = lax.fori_loop(0, L, body, (z, z))
        out_v[i, pl.ds(0, 16)] = a0
        out_v[i, pl.ds(16, 16)] = a1

    fire(0, rows0, sem0)
    fire(1, rows1, sem1)

    def loop_body(j, _):
        i = j * 2
        drain(i, rows0, sem0)
        accumulate(i, rows0)

        @pl.when(i + 2 < BPW)
        def _fire0():
            fire(i + 2, rows0, sem0)

        drain(i + 1, rows1, sem1)
        accumulate(i + 1, rows1)

        @pl.when(i + 3 < BPW)
        def _fire1():
            fire(i + 3, rows1, sem1)

        return 0

    lax.fori_loop(0, BPW // 2, loop_body, 0)

    pltpu.sync_copy(out_v, out_hbm.at[pl.ds(base, BPW)])


def _linear_body(pooled_ref, len_ref, w_ref, b_ref, out_ref):
    x = pooled_ref[...] / len_ref[...].astype(jnp.float32)
    out_ref[...] = (
        lax.dot_general(
            x, w_ref[...], (((1,), (1,)), ((), ())),
            preferred_element_type=jnp.float32,
        )
        + b_ref[...]
    )


_linear = pl.pallas_call(
    _linear_body,
    out_shape=jax.ShapeDtypeStruct((B, OUT_DIM), jnp.float32),
)


def kernel(data, length, table, W, b):
    data = data.astype(jnp.int32)
    pooled = _pool(data, table)
    return _linear(pooled, length.reshape(B, 1), W, b.reshape(1, OUT_DIM))
